# baseline (speedup 1.0000x reference)
import functools

import jax
import jax.numpy as jnp
from jax import lax
from jax.experimental import pallas as pl
from jax.experimental.pallas import tpu as pltpu
from jax.experimental.pallas import tpu_sc as plsc

N = 10000
E = 80000
D = 512
NT = 16
EPT = E // NT
BLK = 128
NBLK = (EPT + BLK - 1) // BLK
EPAD = NBLK * BLK
NPAD = 10112
ROWS_Z = NPAD // NT
NCHUNK = 4


QBLK = BLK
QNB = EPAD // QBLK


def _prep_edges(ei):
  src = ei[0].reshape(NT, EPT)
  dst = ei[1].reshape(NT, EPT)
  pad = EPAD - EPT
  tix = jnp.arange(NT, dtype=jnp.int32)[:, None]
  src = jnp.concatenate(
      [src, jnp.broadcast_to(tix * 625, (NT, pad))], axis=1)
  dst = jnp.concatenate(
      [dst, jnp.broadcast_to(N + tix, (NT, pad))], axis=1)
  return src.reshape(NT, 1, EPAD), dst.reshape(NT, 1, EPAD)


def _seg_mesh():
  return plsc.VectorSubcoreMesh(core_axis_name="c", subcore_axis_name="s")


SBLK = 256
SNB = EPAD // SBLK


def _k1_body(xc0, xc1, xc2, xc3, src1_hbm, dst1_hbm, zf_hbm, z1_hbm,
             s_out, cnt0_out, cnt1_out,
             idx_s, idx_d, buf, ones_b, acc, cacc, sem0, sem1):
  c = lax.axis_index("c")
  s = lax.axis_index("s")

  pltpu.sync_copy(src1_hbm.at[c, s, 0], idx_s)
  pltpu.sync_copy(dst1_hbm.at[c, s, 0], idx_d)

  def _init_ones(i, carry):
    ones_b[pl.ds(i * 16, 16)] = jnp.full((16,), 1.0, jnp.float32)
    return carry
  lax.fori_loop(0, EPAD // 16, _init_ones, 0)

  @pl.when(s == 0)
  def _():
    pltpu.sync_copy(z1_hbm, cacc)
  plsc.subcore_barrier()

  pltpu.sync_copy(ones_b, cacc.at[idx_d], add=True)

  plsc.subcore_barrier()

  @pl.when(jnp.logical_and(s == 0, c == 0))
  def _():
    pltpu.sync_copy(cacc, cnt0_out)

  @pl.when(jnp.logical_and(s == 0, c == 1))
  def _():
    pltpu.sync_copy(cacc, cnt1_out)

  for d, xr in enumerate((xc0, xc1, xc2, xc3)):
    plsc.subcore_barrier()
    pltpu.sync_copy(zf_hbm.at[pl.ds(s * ROWS_Z, ROWS_Z)],
                    acc.at[pl.ds(s * ROWS_Z, ROWS_Z)])
    plsc.subcore_barrier()

    def _blk_step(j, carry):
      pltpu.async_copy(xr.at[idx_s.at[pl.ds(j * SBLK, BLK)]],
                       buf.at[pl.ds(0, BLK)], sem0)
      pltpu.async_copy(xr.at[idx_s.at[pl.ds(j * SBLK + BLK, BLK)]],
                       buf.at[pl.ds(BLK, BLK)], sem1)
      pltpu.make_async_copy(xr.at[idx_s.at[pl.ds(j * SBLK, BLK)]],
                            buf.at[pl.ds(0, BLK)], sem0).wait()
      pltpu.make_async_copy(xr.at[idx_s.at[pl.ds(j * SBLK + BLK, BLK)]],
                            buf.at[pl.ds(BLK, BLK)], sem1).wait()
      pltpu.sync_copy(buf, acc.at[idx_d.at[pl.ds(j * SBLK, SBLK)]], add=True)
      return carry
    lax.fori_loop(0, SNB, _blk_step, 0)

    plsc.subcore_barrier()
    pltpu.sync_copy(acc.at[pl.ds(s * ROWS_Z, ROWS_Z)],
                    s_out.at[c, d, pl.ds(s * ROWS_Z, ROWS_Z)])


def _k1(xcs, src1_hbm, dst1_hbm, zf, z1):
  f = pl.kernel(
      _k1_body,
      out_type=(
          jax.ShapeDtypeStruct((2, NCHUNK, NPAD, BLK), jnp.float32),
          jax.ShapeDtypeStruct((NPAD,), jnp.float32),
          jax.ShapeDtypeStruct((NPAD,), jnp.float32),
      ),
      mesh=_seg_mesh(),
      scratch_types=[
          pltpu.VMEM((EPAD,), jnp.int32),
          pltpu.VMEM((EPAD,), jnp.int32),
          pltpu.VMEM((SBLK, BLK), jnp.float32),
          pltpu.VMEM((EPAD,), jnp.float32),
          pltpu.VMEM_SHARED((NPAD, BLK), jnp.float32),
          pltpu.VMEM_SHARED((NPAD,), jnp.float32),
          pltpu.SemaphoreType.DMA,
          pltpu.SemaphoreType.DMA,
      ],
  )
  return f(*xcs, src1_hbm, dst1_hbm, zf, z1)


def _k3_body(s0, s1, src1_hbm, dst1_hbm, z1_hbm, t0_out, t1_out,
             idx_s, idx_d, val_buf, tacc, sem0):
  c = lax.axis_index("c")
  s = lax.axis_index("s")

  pltpu.sync_copy(src1_hbm.at[c, s, 0], idx_s)
  pltpu.sync_copy(dst1_hbm.at[c, s, 0], idx_d)

  @pl.when(s == 0)
  def _():
    pltpu.sync_copy(z1_hbm, tacc)

  @pl.when(c == 0)
  def _():
    pltpu.async_copy(s0.at[idx_s], val_buf, sem0).wait()

  @pl.when(c == 1)
  def _():
    pltpu.async_copy(s1.at[idx_s], val_buf, sem0).wait()

  plsc.subcore_barrier()
  pltpu.sync_copy(val_buf, tacc.at[idx_d], add=True)
  plsc.subcore_barrier()

  @pl.when(jnp.logical_and(s == 0, c == 0))
  def _():
    pltpu.sync_copy(tacc, t0_out)

  @pl.when(jnp.logical_and(s == 0, c == 1))
  def _():
    pltpu.sync_copy(tacc, t1_out)


def _k3(s0, s1, src1_hbm, dst1_hbm, z1):
  f = pl.kernel(
      _k3_body,
      out_type=(
          jax.ShapeDtypeStruct((NPAD,), jnp.float32),
          jax.ShapeDtypeStruct((NPAD,), jnp.float32),
      ),
      mesh=_seg_mesh(),
      scratch_types=[
          pltpu.VMEM((EPAD,), jnp.int32),
          pltpu.VMEM((EPAD,), jnp.int32),
          pltpu.VMEM((EPAD,), jnp.float32),
          pltpu.VMEM_SHARED((NPAD,), jnp.float32),
          pltpu.SemaphoreType.DMA,
      ],
  )
  return f(s0, s1, src1_hbm, dst1_hbm, z1)


_K2_ROWS = 400


def _k2_kernel(x_ref, s00, s01, s02, s03, s10, s11, s12, s13,
               cnt0_ref, cnt1_ref, wl0_ref, wl1_ref, wrs_ref, wcat_ref,
               b1_ref, out_ref):
  inv0 = 1.0 / jnp.maximum(cnt0_ref[...], 1.0)
  inv1 = 1.0 / jnp.maximum(cnt1_ref[...], 1.0)
  acc = jnp.dot(x_ref[...], wrs_ref[...], preferred_element_type=jnp.float32)
  for d, s_ref in enumerate((s00, s01, s02, s03)):
    acc += jnp.dot(s_ref[0, 0] * inv0, wl0_ref[d * BLK:(d + 1) * BLK, :],
                   preferred_element_type=jnp.float32)
  for d, s_ref in enumerate((s10, s11, s12, s13)):
    acc += jnp.dot(s_ref[0, 0] * inv1, wl1_ref[d * BLK:(d + 1) * BLK, :],
                   preferred_element_type=jnp.float32)
  h = jnp.maximum(acc + b1_ref[...], 0.0)
  out_ref[...] = jnp.dot(h, wcat_ref[...], preferred_element_type=jnp.float32)


def _k2(x, s_all, cnt0, cnt1, wl0, wl1, wrs, wcat, b1):
  grid = (N // _K2_ROWS,)
  s_specs = [
      pl.BlockSpec((1, 1, _K2_ROWS, BLK),
                   functools.partial(lambda i, c, d: (c, d, i, 0), c=c, d=d))
      for c in range(2) for d in range(NCHUNK)
  ]
  cnt_specs = [pl.BlockSpec((_K2_ROWS, 1), lambda i: (i, 0))] * 2
  return pl.pallas_call(
      _k2_kernel,
      grid=grid,
      in_specs=[pl.BlockSpec((_K2_ROWS, D), lambda i: (i, 0))] + s_specs
      + cnt_specs + [
          pl.BlockSpec((D, D), lambda i: (0, 0)),
          pl.BlockSpec((D, D), lambda i: (0, 0)),
          pl.BlockSpec((D, D), lambda i: (0, 0)),
          pl.BlockSpec((D, BLK), lambda i: (0, 0)),
          pl.BlockSpec((1, D), lambda i: (0, 0)),
      ],
      out_specs=pl.BlockSpec((_K2_ROWS, BLK), lambda i: (i, 0)),
      out_shape=jax.ShapeDtypeStruct((N, BLK), jnp.float32),
  )(x, *([s_all] * 8), cnt0, cnt1, wl0, wl1, wrs, wcat, b1)


def _k4_kernel(t0_ref, t1_ref, cnt0_ref, cnt1_ref, r_ref, b2_ref, out_ref):
  c0 = jnp.maximum(cnt0_ref[...], 1.0)
  c1 = jnp.maximum(cnt1_ref[...], 1.0)
  o = t0_ref[...] / c0 + t1_ref[...] / c1 + r_ref[...] + b2_ref[0, 0]
  out_ref[...] = jax.nn.sigmoid(o)


def _k4(t0, t1, cnt0, cnt1, r, b2):
  full = pl.BlockSpec((NPAD // BLK, BLK), lambda: (0, 0))
  return pl.pallas_call(
      _k4_kernel,
      in_specs=[full, full, full, full, full,
                pl.BlockSpec((1, 1), lambda: (0, 0), memory_space=pltpu.SMEM)],
      out_specs=full,
      out_shape=jax.ShapeDtypeStruct((NPAD // BLK, BLK), jnp.float32),
  )(t0, t1, cnt0, cnt1, r, b2)


def kernel(x, edge_index_e0, edge_index_e1,
           Wl1_e0, bl1_e0, Wr1_e0, Wl1_e1, bl1_e1, Wr1_e1,
           Wl2_e0, bl2_e0, Wr2_e0, Wl2_e1, bl2_e1, Wr2_e1):
  xcs = tuple(x[:, d * BLK:(d + 1) * BLK] for d in range(NCHUNK))
  s0f, d0f = _prep_edges(edge_index_e0)
  s1f, d1f = _prep_edges(edge_index_e1)
  src1_hbm = jnp.stack([s0f, s1f])
  dst1_hbm = jnp.stack([d0f, d1f])
  zf = jnp.zeros((NPAD, BLK), jnp.float32)

  wrs1 = Wr1_e0 + Wr1_e1
  b1 = (bl1_e0 + bl1_e1).reshape(1, D)
  wcat = jnp.concatenate(
      [Wl2_e0, Wl2_e1, Wr2_e0 + Wr2_e1,
       jnp.zeros((D, BLK - 3), jnp.float32)], axis=1)
  b2 = (bl2_e0 + bl2_e1).reshape(1, 1)

  z1 = jnp.zeros((NPAD,), jnp.float32)
  s_all, cnt0, cnt1 = _k1(xcs, src1_hbm, dst1_hbm, zf, z1)

  g = _k2(x, s_all, cnt0[:N, None], cnt1[:N, None], Wl1_e0, Wl1_e1,
          wrs1, wcat, b1)

  shp = (NPAD // BLK, BLK)
  t0, t1 = _k3(g[:, 0], g[:, 1], src1_hbm, dst1_hbm, z1)

  r = jnp.pad(g[:, 2], (0, NPAD - N)).reshape(shp)
  out = _k4(t0.reshape(shp), t1.reshape(shp),
            cnt0.reshape(shp), cnt1.reshape(shp), r, b2)
  return out.reshape(NPAD)[:N, None]


# speedup vs baseline: 1.1853x; 1.1853x over previous
import functools

import jax
import jax.numpy as jnp
from jax import lax
from jax.experimental import pallas as pl
from jax.experimental.pallas import tpu as pltpu
from jax.experimental.pallas import tpu_sc as plsc

N = 10000
E = 80000
D = 512
NT = 16
EPT = E // NT
BLK = 128
NBLK = (EPT + BLK - 1) // BLK
EPAD = NBLK * BLK
NPAD = 10112
ROWS_Z = NPAD // NT
NCHUNK = 4


QBLK = BLK
QNB = EPAD // QBLK


def _prep_edges(ei):
  src = ei[0].reshape(NT, EPT)
  dst = ei[1].reshape(NT, EPT)
  pad = EPAD - EPT
  tix = jnp.arange(NT, dtype=jnp.int32)[:, None]
  src = jnp.concatenate(
      [src, jnp.broadcast_to(tix * 625, (NT, pad))], axis=1)
  dst = jnp.concatenate(
      [dst, jnp.broadcast_to(N + tix, (NT, pad))], axis=1)
  return src.reshape(NT, 1, EPAD), dst.reshape(NT, 1, EPAD)


def _seg_mesh():
  return plsc.VectorSubcoreMesh(core_axis_name="c", subcore_axis_name="s")


SBLK = 256
SNB = EPAD // SBLK


def _k1_body(xc0, xc1, xc2, xc3, src1_hbm, dst1_hbm, zf_hbm, z1_hbm,
             s_out, cnt0_out, cnt1_out,
             idx_s, idx_d, buf, ones_b, acc, cacc, sem0, sem1):
  c = lax.axis_index("c")
  s = lax.axis_index("s")

  pltpu.sync_copy(src1_hbm.at[c, s, 0], idx_s)
  pltpu.sync_copy(dst1_hbm.at[c, s, 0], idx_d)

  def _init_ones(i, carry):
    ones_b[pl.ds(i * 16, 16)] = jnp.full((16,), 1.0, jnp.float32)
    return carry
  lax.fori_loop(0, EPAD // 16, _init_ones, 0)

  @pl.when(s == 0)
  def _():
    pltpu.sync_copy(z1_hbm, cacc)
  plsc.subcore_barrier()

  pltpu.sync_copy(ones_b, cacc.at[idx_d], add=True)

  plsc.subcore_barrier()

  @pl.when(jnp.logical_and(s == 0, c == 0))
  def _():
    pltpu.sync_copy(cacc, cnt0_out)

  @pl.when(jnp.logical_and(s == 0, c == 1))
  def _():
    pltpu.sync_copy(cacc, cnt1_out)

  for d, xr in enumerate((xc0, xc1, xc2, xc3)):
    plsc.subcore_barrier()
    pltpu.sync_copy(zf_hbm.at[pl.ds(s * ROWS_Z, ROWS_Z)],
                    acc.at[pl.ds(s * ROWS_Z, ROWS_Z)])
    plsc.subcore_barrier()

    b0 = buf.at[pl.ds(0, BLK)]
    b1 = buf.at[pl.ds(BLK, BLK)]
    pltpu.async_copy(xr.at[idx_s.at[pl.ds(0, BLK)]], b0, sem0)
    pltpu.async_copy(xr.at[idx_s.at[pl.ds(BLK, BLK)]], b1, sem1)

    def _blk_step(t, carry):
      for b, (bref, sem) in enumerate(((b0, sem0), (b1, sem1))):
        j = 2 * t + b
        pltpu.make_async_copy(xr.at[idx_s.at[pl.ds(j * BLK, BLK)]],
                              bref, sem).wait()
        pltpu.sync_copy(bref, acc.at[idx_d.at[pl.ds(j * BLK, BLK)]], add=True)

        @pl.when(j + 2 < QNB)
        def _():
          pltpu.async_copy(xr.at[idx_s.at[pl.ds((j + 2) * BLK, BLK)]],
                           bref, sem)
      return carry
    lax.fori_loop(0, QNB // 2, _blk_step, 0)

    plsc.subcore_barrier()
    pltpu.sync_copy(acc.at[pl.ds(s * ROWS_Z, ROWS_Z)],
                    s_out.at[c, d, pl.ds(s * ROWS_Z, ROWS_Z)])


def _k1(xcs, src1_hbm, dst1_hbm, zf, z1):
  f = pl.kernel(
      _k1_body,
      out_type=(
          jax.ShapeDtypeStruct((2, NCHUNK, NPAD, BLK), jnp.float32),
          jax.ShapeDtypeStruct((NPAD,), jnp.float32),
          jax.ShapeDtypeStruct((NPAD,), jnp.float32),
      ),
      mesh=_seg_mesh(),
      scratch_types=[
          pltpu.VMEM((EPAD,), jnp.int32),
          pltpu.VMEM((EPAD,), jnp.int32),
          pltpu.VMEM((SBLK, BLK), jnp.float32),
          pltpu.VMEM((EPAD,), jnp.float32),
          pltpu.VMEM_SHARED((NPAD, BLK), jnp.float32),
          pltpu.VMEM_SHARED((NPAD,), jnp.float32),
          pltpu.SemaphoreType.DMA,
          pltpu.SemaphoreType.DMA,
      ],
  )
  return f(*xcs, src1_hbm, dst1_hbm, zf, z1)


def _k3_body(s0, s1, src1_hbm, dst1_hbm, z1_hbm, t0_out, t1_out,
             idx_s, idx_d, val_buf, tacc, sem0):
  c = lax.axis_index("c")
  s = lax.axis_index("s")

  pltpu.sync_copy(src1_hbm.at[c, s, 0], idx_s)
  pltpu.sync_copy(dst1_hbm.at[c, s, 0], idx_d)

  @pl.when(s == 0)
  def _():
    pltpu.sync_copy(z1_hbm, tacc)

  @pl.when(c == 0)
  def _():
    pltpu.async_copy(s0.at[idx_s], val_buf, sem0).wait()

  @pl.when(c == 1)
  def _():
    pltpu.async_copy(s1.at[idx_s], val_buf, sem0).wait()

  plsc.subcore_barrier()
  pltpu.sync_copy(val_buf, tacc.at[idx_d], add=True)
  plsc.subcore_barrier()

  @pl.when(jnp.logical_and(s == 0, c == 0))
  def _():
    pltpu.sync_copy(tacc, t0_out)

  @pl.when(jnp.logical_and(s == 0, c == 1))
  def _():
    pltpu.sync_copy(tacc, t1_out)


def _k3(s0, s1, src1_hbm, dst1_hbm, z1):
  f = pl.kernel(
      _k3_body,
      out_type=(
          jax.ShapeDtypeStruct((NPAD,), jnp.float32),
          jax.ShapeDtypeStruct((NPAD,), jnp.float32),
      ),
      mesh=_seg_mesh(),
      scratch_types=[
          pltpu.VMEM((EPAD,), jnp.int32),
          pltpu.VMEM((EPAD,), jnp.int32),
          pltpu.VMEM((EPAD,), jnp.float32),
          pltpu.VMEM_SHARED((NPAD,), jnp.float32),
          pltpu.SemaphoreType.DMA,
      ],
  )
  return f(s0, s1, src1_hbm, dst1_hbm, z1)


_K2_ROWS = 400


def _k2_kernel(x_ref, s00, s01, s02, s03, s10, s11, s12, s13,
               cnt0_ref, cnt1_ref, wl0_ref, wl1_ref, wrs_ref, wcat_ref,
               b1_ref, out_ref):
  inv0 = 1.0 / jnp.maximum(cnt0_ref[...], 1.0)
  inv1 = 1.0 / jnp.maximum(cnt1_ref[...], 1.0)
  acc = jnp.dot(x_ref[...], wrs_ref[...], preferred_element_type=jnp.float32)
  for d, s_ref in enumerate((s00, s01, s02, s03)):
    acc += jnp.dot(s_ref[0, 0] * inv0, wl0_ref[d * BLK:(d + 1) * BLK, :],
                   preferred_element_type=jnp.float32)
  for d, s_ref in enumerate((s10, s11, s12, s13)):
    acc += jnp.dot(s_ref[0, 0] * inv1, wl1_ref[d * BLK:(d + 1) * BLK, :],
                   preferred_element_type=jnp.float32)
  h = jnp.maximum(acc + b1_ref[...], 0.0)
  out_ref[...] = jnp.dot(h, wcat_ref[...], preferred_element_type=jnp.float32)


def _k2(x, s_all, cnt0, cnt1, wl0, wl1, wrs, wcat, b1):
  grid = (N // _K2_ROWS,)
  s_specs = [
      pl.BlockSpec((1, 1, _K2_ROWS, BLK),
                   functools.partial(lambda i, c, d: (c, d, i, 0), c=c, d=d))
      for c in range(2) for d in range(NCHUNK)
  ]
  cnt_specs = [pl.BlockSpec((_K2_ROWS, 1), lambda i: (i, 0))] * 2
  return pl.pallas_call(
      _k2_kernel,
      grid=grid,
      in_specs=[pl.BlockSpec((_K2_ROWS, D), lambda i: (i, 0))] + s_specs
      + cnt_specs + [
          pl.BlockSpec((D, D), lambda i: (0, 0)),
          pl.BlockSpec((D, D), lambda i: (0, 0)),
          pl.BlockSpec((D, D), lambda i: (0, 0)),
          pl.BlockSpec((D, BLK), lambda i: (0, 0)),
          pl.BlockSpec((1, D), lambda i: (0, 0)),
      ],
      out_specs=pl.BlockSpec((_K2_ROWS, BLK), lambda i: (i, 0)),
      out_shape=jax.ShapeDtypeStruct((N, BLK), jnp.float32),
  )(x, *([s_all] * 8), cnt0, cnt1, wl0, wl1, wrs, wcat, b1)


def _k4_kernel(t0_ref, t1_ref, cnt0_ref, cnt1_ref, r_ref, b2_ref, out_ref):
  c0 = jnp.maximum(cnt0_ref[...], 1.0)
  c1 = jnp.maximum(cnt1_ref[...], 1.0)
  o = t0_ref[...] / c0 + t1_ref[...] / c1 + r_ref[...] + b2_ref[0, 0]
  out_ref[...] = jax.nn.sigmoid(o)


def _k4(t0, t1, cnt0, cnt1, r, b2):
  full = pl.BlockSpec((NPAD // BLK, BLK), lambda: (0, 0))
  return pl.pallas_call(
      _k4_kernel,
      in_specs=[full, full, full, full, full,
                pl.BlockSpec((1, 1), lambda: (0, 0), memory_space=pltpu.SMEM)],
      out_specs=full,
      out_shape=jax.ShapeDtypeStruct((NPAD // BLK, BLK), jnp.float32),
  )(t0, t1, cnt0, cnt1, r, b2)


def kernel(x, edge_index_e0, edge_index_e1,
           Wl1_e0, bl1_e0, Wr1_e0, Wl1_e1, bl1_e1, Wr1_e1,
           Wl2_e0, bl2_e0, Wr2_e0, Wl2_e1, bl2_e1, Wr2_e1):
  xcs = tuple(x[:, d * BLK:(d + 1) * BLK] for d in range(NCHUNK))
  s0f, d0f = _prep_edges(edge_index_e0)
  s1f, d1f = _prep_edges(edge_index_e1)
  src1_hbm = jnp.stack([s0f, s1f])
  dst1_hbm = jnp.stack([d0f, d1f])
  zf = jnp.zeros((NPAD, BLK), jnp.float32)

  wrs1 = Wr1_e0 + Wr1_e1
  b1 = (bl1_e0 + bl1_e1).reshape(1, D)
  wcat = jnp.concatenate(
      [Wl2_e0, Wl2_e1, Wr2_e0 + Wr2_e1,
       jnp.zeros((D, BLK - 3), jnp.float32)], axis=1)
  b2 = (bl2_e0 + bl2_e1).reshape(1, 1)

  z1 = jnp.zeros((NPAD,), jnp.float32)
  s_all, cnt0, cnt1 = _k1(xcs, src1_hbm, dst1_hbm, zf, z1)

  g = _k2(x, s_all, cnt0[:N, None], cnt1[:N, None], Wl1_e0, Wl1_e1,
          wrs1, wcat, b1)

  shp = (NPAD // BLK, BLK)
  t0, t1 = _k3(g[:, 0], g[:, 1], src1_hbm, dst1_hbm, z1)

  r = jnp.pad(g[:, 2], (0, NPAD - N)).reshape(shp)
  out = _k4(t0.reshape(shp), t1.reshape(shp),
            cnt0.reshape(shp), cnt1.reshape(shp), r, b2)
  return out.reshape(NPAD)[:N, None]


# speedup vs baseline: 1.1901x; 1.0040x over previous
import functools

import jax
import jax.numpy as jnp
from jax import lax
from jax.experimental import pallas as pl
from jax.experimental.pallas import tpu as pltpu
from jax.experimental.pallas import tpu_sc as plsc

N = 10000
E = 80000
D = 512
NT = 16
EPT = E // NT
BLK = 128
NBLK = (EPT + BLK - 1) // BLK
EPAD = NBLK * BLK
NPAD = 10112
ROWS_Z = NPAD // NT
NCHUNK = 4


QBLK = BLK
QNB = EPAD // QBLK


def _prep_edges(ei):
  src = ei[0].reshape(NT, EPT)
  dst = ei[1].reshape(NT, EPT)
  pad = EPAD - EPT
  tix = jnp.arange(NT, dtype=jnp.int32)[:, None]
  src = jnp.concatenate(
      [src, jnp.broadcast_to(tix * 625, (NT, pad))], axis=1)
  dst = jnp.concatenate(
      [dst, jnp.broadcast_to(N + tix, (NT, pad))], axis=1)
  return src.reshape(NT, 1, EPAD), dst.reshape(NT, 1, EPAD)


def _seg_mesh():
  return plsc.VectorSubcoreMesh(core_axis_name="c", subcore_axis_name="s")


SBLK = 256
SNB = EPAD // SBLK


def _k1_body(x_hbm, src1_hbm, dst1_hbm, zf_hbm, z1_hbm,
             s_out, cnt0_out, cnt1_out,
             idx_s, idx_d, buf, ones_b, acc, cacc, sem0, sem1):
  c = lax.axis_index("c")
  s = lax.axis_index("s")

  pltpu.sync_copy(src1_hbm.at[c, s, 0], idx_s)
  pltpu.sync_copy(dst1_hbm.at[c, s, 0], idx_d)

  def _init_ones(i, carry):
    ones_b[pl.ds(i * 16, 16)] = jnp.full((16,), 1.0, jnp.float32)
    return carry
  lax.fori_loop(0, EPAD // 16, _init_ones, 0)

  @pl.when(s == 0)
  def _():
    pltpu.sync_copy(z1_hbm, cacc)
  plsc.subcore_barrier()

  pltpu.sync_copy(ones_b, cacc.at[idx_d], add=True)

  plsc.subcore_barrier()

  @pl.when(jnp.logical_and(s == 0, c == 0))
  def _():
    pltpu.sync_copy(cacc, cnt0_out)

  @pl.when(jnp.logical_and(s == 0, c == 1))
  def _():
    pltpu.sync_copy(cacc, cnt1_out)

  for d in range(NCHUNK):
    xr = x_hbm.at[:, pl.ds(d * BLK, BLK)]
    pltpu.sync_copy(zf_hbm, acc.at[pl.ds(s * ROWS_Z, ROWS_Z)])
    plsc.subcore_barrier()

    b0 = buf.at[pl.ds(0, BLK)]
    b1 = buf.at[pl.ds(BLK, BLK)]
    pltpu.async_copy(xr.at[idx_s.at[pl.ds(0, BLK)]], b0, sem0)
    pltpu.async_copy(xr.at[idx_s.at[pl.ds(BLK, BLK)]], b1, sem1)

    def _blk_step(t, carry):
      for b, (bref, sem) in enumerate(((b0, sem0), (b1, sem1))):
        j = 2 * t + b
        pltpu.make_async_copy(xr.at[idx_s.at[pl.ds(j * BLK, BLK)]],
                              bref, sem).wait()
        pltpu.sync_copy(bref, acc.at[idx_d.at[pl.ds(j * BLK, BLK)]], add=True)

        @pl.when(j + 2 < QNB)
        def _():
          pltpu.async_copy(xr.at[idx_s.at[pl.ds((j + 2) * BLK, BLK)]],
                           bref, sem)
      return carry
    lax.fori_loop(0, QNB // 2, _blk_step, 0)

    plsc.subcore_barrier()
    pltpu.sync_copy(acc.at[pl.ds(s * ROWS_Z, ROWS_Z)],
                    s_out.at[c, d, pl.ds(s * ROWS_Z, ROWS_Z)])


def _k1(x, src1_hbm, dst1_hbm, zf, z1):
  f = pl.kernel(
      _k1_body,
      out_type=(
          jax.ShapeDtypeStruct((2, NCHUNK, NPAD, BLK), jnp.float32),
          jax.ShapeDtypeStruct((NPAD,), jnp.float32),
          jax.ShapeDtypeStruct((NPAD,), jnp.float32),
      ),
      mesh=_seg_mesh(),
      scratch_types=[
          pltpu.VMEM((EPAD,), jnp.int32),
          pltpu.VMEM((EPAD,), jnp.int32),
          pltpu.VMEM((SBLK, BLK), jnp.float32),
          pltpu.VMEM((EPAD,), jnp.float32),
          pltpu.VMEM_SHARED((NPAD, BLK), jnp.float32),
          pltpu.VMEM_SHARED((NPAD,), jnp.float32),
          pltpu.SemaphoreType.DMA,
          pltpu.SemaphoreType.DMA,
      ],
  )
  return f(x, src1_hbm, dst1_hbm, zf, z1)


def _k3_body(s0, s1, src1_hbm, dst1_hbm, z1_hbm, t0_out, t1_out,
             idx_s, idx_d, val_buf, tacc, sem0):
  c = lax.axis_index("c")
  s = lax.axis_index("s")

  pltpu.sync_copy(src1_hbm.at[c, s, 0], idx_s)
  pltpu.sync_copy(dst1_hbm.at[c, s, 0], idx_d)

  @pl.when(s == 0)
  def _():
    pltpu.sync_copy(z1_hbm, tacc)

  @pl.when(c == 0)
  def _():
    pltpu.async_copy(s0.at[idx_s], val_buf, sem0).wait()

  @pl.when(c == 1)
  def _():
    pltpu.async_copy(s1.at[idx_s], val_buf, sem0).wait()

  plsc.subcore_barrier()
  pltpu.sync_copy(val_buf, tacc.at[idx_d], add=True)
  plsc.subcore_barrier()

  @pl.when(jnp.logical_and(s == 0, c == 0))
  def _():
    pltpu.sync_copy(tacc, t0_out)

  @pl.when(jnp.logical_and(s == 0, c == 1))
  def _():
    pltpu.sync_copy(tacc, t1_out)


def _k3(s0, s1, src1_hbm, dst1_hbm, z1):
  f = pl.kernel(
      _k3_body,
      out_type=(
          jax.ShapeDtypeStruct((NPAD,), jnp.float32),
          jax.ShapeDtypeStruct((NPAD,), jnp.float32),
      ),
      mesh=_seg_mesh(),
      scratch_types=[
          pltpu.VMEM((EPAD,), jnp.int32),
          pltpu.VMEM((EPAD,), jnp.int32),
          pltpu.VMEM((EPAD,), jnp.float32),
          pltpu.VMEM_SHARED((NPAD,), jnp.float32),
          pltpu.SemaphoreType.DMA,
      ],
  )
  return f(s0, s1, src1_hbm, dst1_hbm, z1)


_K2_ROWS = 400


def _k2_kernel(x_ref, s00, s01, s02, s03, s10, s11, s12, s13,
               cnt0_ref, cnt1_ref, wl0_ref, wl1_ref, wrs_ref, wcat_ref,
               b1_ref, out_ref):
  inv0 = 1.0 / jnp.maximum(cnt0_ref[...], 1.0)
  inv1 = 1.0 / jnp.maximum(cnt1_ref[...], 1.0)
  acc = jnp.dot(x_ref[...], wrs_ref[...], preferred_element_type=jnp.float32)
  for d, s_ref in enumerate((s00, s01, s02, s03)):
    acc += jnp.dot(s_ref[0, 0] * inv0, wl0_ref[d * BLK:(d + 1) * BLK, :],
                   preferred_element_type=jnp.float32)
  for d, s_ref in enumerate((s10, s11, s12, s13)):
    acc += jnp.dot(s_ref[0, 0] * inv1, wl1_ref[d * BLK:(d + 1) * BLK, :],
                   preferred_element_type=jnp.float32)
  h = jnp.maximum(acc + b1_ref[...], 0.0)
  out_ref[...] = jnp.dot(h, wcat_ref[...], preferred_element_type=jnp.float32)


def _k2(x, s_all, cnt0, cnt1, wl0, wl1, wrs, wcat, b1):
  grid = (N // _K2_ROWS,)
  s_specs = [
      pl.BlockSpec((1, 1, _K2_ROWS, BLK),
                   functools.partial(lambda i, c, d: (c, d, i, 0), c=c, d=d))
      for c in range(2) for d in range(NCHUNK)
  ]
  cnt_specs = [pl.BlockSpec((_K2_ROWS, 1), lambda i: (i, 0))] * 2
  return pl.pallas_call(
      _k2_kernel,
      grid=grid,
      in_specs=[pl.BlockSpec((_K2_ROWS, D), lambda i: (i, 0))] + s_specs
      + cnt_specs + [
          pl.BlockSpec((D, D), lambda i: (0, 0)),
          pl.BlockSpec((D, D), lambda i: (0, 0)),
          pl.BlockSpec((D, D), lambda i: (0, 0)),
          pl.BlockSpec((D, BLK), lambda i: (0, 0)),
          pl.BlockSpec((1, D), lambda i: (0, 0)),
      ],
      out_specs=pl.BlockSpec((_K2_ROWS, BLK), lambda i: (i, 0)),
      out_shape=jax.ShapeDtypeStruct((N, BLK), jnp.float32),
  )(x, *([s_all] * 8), cnt0, cnt1, wl0, wl1, wrs, wcat, b1)


def _k4_kernel(t0_ref, t1_ref, cnt0_ref, cnt1_ref, r_ref, b2_ref, out_ref):
  c0 = jnp.maximum(cnt0_ref[...], 1.0)
  c1 = jnp.maximum(cnt1_ref[...], 1.0)
  o = t0_ref[...] / c0 + t1_ref[...] / c1 + r_ref[...] + b2_ref[0, 0]
  out_ref[...] = jax.nn.sigmoid(o)


def _k4(t0, t1, cnt0, cnt1, r, b2):
  full = pl.BlockSpec((NPAD // BLK, BLK), lambda: (0, 0))
  return pl.pallas_call(
      _k4_kernel,
      in_specs=[full, full, full, full, full,
                pl.BlockSpec((1, 1), lambda: (0, 0), memory_space=pltpu.SMEM)],
      out_specs=full,
      out_shape=jax.ShapeDtypeStruct((NPAD // BLK, BLK), jnp.float32),
  )(t0, t1, cnt0, cnt1, r, b2)


def kernel(x, edge_index_e0, edge_index_e1,
           Wl1_e0, bl1_e0, Wr1_e0, Wl1_e1, bl1_e1, Wr1_e1,
           Wl2_e0, bl2_e0, Wr2_e0, Wl2_e1, bl2_e1, Wr2_e1):
  s0f, d0f = _prep_edges(edge_index_e0)
  s1f, d1f = _prep_edges(edge_index_e1)
  src1_hbm = jnp.stack([s0f, s1f])
  dst1_hbm = jnp.stack([d0f, d1f])
  zf = jnp.zeros((ROWS_Z, BLK), jnp.float32)

  wrs1 = Wr1_e0 + Wr1_e1
  b1 = (bl1_e0 + bl1_e1).reshape(1, D)
  wcat = jnp.concatenate(
      [Wl2_e0, Wl2_e1, Wr2_e0 + Wr2_e1,
       jnp.zeros((D, BLK - 3), jnp.float32)], axis=1)
  b2 = (bl2_e0 + bl2_e1).reshape(1, 1)

  z1 = jnp.zeros((NPAD,), jnp.float32)
  s_all, cnt0, cnt1 = _k1(x, src1_hbm, dst1_hbm, zf, z1)

  g = _k2(x, s_all, cnt0[:N, None], cnt1[:N, None], Wl1_e0, Wl1_e1,
          wrs1, wcat, b1)

  shp = (NPAD // BLK, BLK)
  t0, t1 = _k3(g[:, 0], g[:, 1], src1_hbm, dst1_hbm, z1)

  r = jnp.pad(g[:, 2], (0, NPAD - N)).reshape(shp)
  out = _k4(t0.reshape(shp), t1.reshape(shp),
            cnt0.reshape(shp), cnt1.reshape(shp), r, b2)
  return out.reshape(NPAD)[:N, None]
